# Initial kernel scaffold; baseline (speedup 1.0000x reference)
import jax
import jax.numpy as jnp
from jax.experimental import pallas as pl


def kernel(feature_maps, pts, cam_intrinsics, cam_extrinsics):
    raise NotImplementedError("write your pallas kernel here")


# speedup vs baseline: 6.1127x; 6.1127x over previous
import functools

import jax
import jax.numpy as jnp
from jax import lax
from jax.experimental import pallas as pl
from jax.experimental.pallas import tpu as pltpu
from jax.experimental.pallas import tpu_sc as plsc

B, V, C, H, W = 2, 4, 16, 256, 256
NB = B * V
NPTS = 50000
NPAD = 50176
QUART = NPAD // 4
CHUNK = 128
NCHUNK = QUART // CHUNK
NGROUP = CHUNK // 16
A = float(W) / float(W - 1)

_SAMPLES = ((0, 0), (1, 0), (2, 0), (0, 1), (0, 2))


def _corner(f):
    fc = jnp.minimum(jnp.maximum(f, -4.0), 300.0)
    ti = fc.astype(jnp.int32)
    tf = ti.astype(jnp.float32)
    neg = tf > fc
    f0 = jnp.where(neg, tf - 1.0, tf)
    i0 = jnp.where(neg, ti - 1, ti)
    w1 = fc - f0
    w0 = 1.0 - w1
    a0 = jnp.where((i0 >= 0) & (i0 <= W - 1), w0, 0.0)
    a1 = jnp.where((i0 >= -1) & (i0 <= W - 2), w1, 0.0)
    return i0, a0, a1


def _tec_body(fm_ref, ixy_ref, pf_ref, gx_ref, gy_ref,
              cxy_v, idx_v, w_v, col_v, rows_v,
              pfb, gxb, gyb, sem):
    cid = lax.axis_index("c")
    sid = lax.axis_index("s")
    wid = sid * 2 + cid
    vw = wid // 4
    qt = wid % 4
    vbase = vw * (H * W)

    def chunk_body(g, carry):
        pbase = qt * QUART + g * CHUNK
        poff = vw * 6 * NPAD + pbase
        for r in range(6):
            pltpu.sync_copy(ixy_ref.at[pl.ds(poff + r * NPAD, CHUNK)],
                            cxy_v.at[pl.ds(r * CHUNK, CHUNK)])

        def p1(gi, c1):
            sl = gi * 16
            xs = tuple(cxy_v[pl.ds(r * CHUNK + sl, 16)] for r in (0, 1, 2))
            ys = tuple(cxy_v[pl.ds(r * CHUNK + sl, 16)] for r in (3, 4, 5))
            xb, xa, xcol = [], [], []
            for f in xs:
                i0, a0, a1 = _corner(f)
                bx = jnp.minimum(jnp.maximum(i0, 0), W - 2)
                match = i0 == bx
                xb.append(bx)
                xa.append((a0, a1))
                xcol.append((jnp.where(match, 0, C), jnp.where(match, C, 0)))
            yb, ya = [], []
            for f in ys:
                i0, a0, a1 = _corner(f)
                yb.append((jnp.minimum(jnp.maximum(i0, 0), H - 1),
                           jnp.minimum(jnp.maximum(i0 + 1, 0), H - 1)))
                ya.append((a0, a1))
            for s, (xv, yv) in enumerate(_SAMPLES):
                for j in (0, 1):
                    q = 2 * s + j
                    idx_v[pl.ds(q * CHUNK + gi * 16, 16)] = (
                        yb[yv][j] * W + xb[xv] + vbase)
                    for i in (0, 1):
                        w_v[pl.ds((s * 4 + j * 2 + i) * CHUNK + gi * 16, 16)] = (
                            xa[xv][i] * ya[yv][j])
            for m, (xv, i) in enumerate(
                    ((0, 0), (0, 1), (1, 0), (1, 1), (2, 0), (2, 1))):
                col_v[pl.ds(m * CHUNK + gi * 16, 16)] = xcol[xv][i]
            return c1

        lax.fori_loop(0, NGROUP, p1, 0)

        descs = [pltpu.async_copy(
            fm_ref.at[idx_v.at[pl.ds(q * CHUNK, CHUNK)]],
            rows_v.at[pl.ds(q * CHUNK, CHUNK)], sem) for q in range(10)]
        for d in descs:
            d.wait()

        def p2(gi, c2):
            sl = pl.ds(gi * 16, 16)
            ws = [w_v[pl.ds(kk * CHUNK + gi * 16, 16)] for kk in range(20)]
            cols = [col_v[pl.ds(m * CHUNK + gi * 16, 16)] for m in range(6)]
            pvec = lax.iota(jnp.int32, 16) + gi * 16
            rowv = [pvec + q * CHUNK for q in range(10)]
            for ch in range(C):
                cc = [c_ + ch for c_ in cols]
                accs = []
                for s, (xv, _) in enumerate(_SAMPLES):
                    q0, q1 = 2 * s, 2 * s + 1
                    m0, m1 = 2 * xv, 2 * xv + 1
                    a = ws[s * 4 + 0] * plsc.load_gather(rows_v, [rowv[q0], cc[m0]])
                    a = a + ws[s * 4 + 1] * plsc.load_gather(rows_v, [rowv[q0], cc[m1]])
                    a = a + ws[s * 4 + 2] * plsc.load_gather(rows_v, [rowv[q1], cc[m0]])
                    a = a + ws[s * 4 + 3] * plsc.load_gather(rows_v, [rowv[q1], cc[m1]])
                    accs.append(a)
                pfb[ch, sl] = accs[0]
                gxb[ch, sl] = (accs[2] - accs[1]) * 0.5
                gyb[ch, sl] = (accs[4] - accs[3]) * 0.5
            return c2

        lax.fori_loop(0, NGROUP, p2, 0)
        rsl = (pl.ds(vw * C, C), pl.ds(pbase, CHUNK))
        pltpu.sync_copy(pfb, pf_ref.at[rsl])
        pltpu.sync_copy(gxb, gx_ref.at[rsl])
        pltpu.sync_copy(gyb, gy_ref.at[rsl])
        return carry

    lax.fori_loop(0, NCHUNK, chunk_body, 0)


_out3 = [jax.ShapeDtypeStruct((NB * C, NPAD), jnp.float32)] * 3

_sc_fetch = pl.kernel(
    _tec_body,
    _out3,
    mesh=plsc.VectorSubcoreMesh(core_axis_name="c", subcore_axis_name="s",
                                num_cores=2, num_subcores=16),
    scratch_types=[
        pltpu.VMEM((6 * CHUNK,), jnp.float32),
        pltpu.VMEM((10 * CHUNK,), jnp.int32),
        pltpu.VMEM((20 * CHUNK,), jnp.float32),
        pltpu.VMEM((6 * CHUNK,), jnp.int32),
        pltpu.VMEM((10 * CHUNK, 2 * C), jnp.float32),
        pltpu.VMEM((C, CHUNK), jnp.float32),
        pltpu.VMEM((C, CHUNK), jnp.float32),
        pltpu.VMEM((C, CHUNK), jnp.float32),
        pltpu.SemaphoreType.DMA,
    ],
    compiler_params=pltpu.CompilerParams(needs_layout_passes=False,
                                         use_tc_tiling_on_sc=False),
)


def kernel(feature_maps, pts, cam_intrinsics, cam_extrinsics):
    a = jnp.transpose(feature_maps.reshape(NB, C, H * W), (0, 2, 1))
    nxt = jnp.concatenate([a[:, 1:, :], a[:, -1:, :]], axis=1)
    fm_pair = jnp.concatenate([a, nxt], axis=2).reshape(NB * H * W, 2 * C)

    K3 = cam_intrinsics.reshape(NB, 3, 3)
    E3 = cam_extrinsics.reshape(NB, 3, 4)
    R = E3[:, :, 0:3]
    t = E3[:, :, 3:4]
    pts_e = jnp.broadcast_to(pts[:, None, :, :],
                             (B, V, 3, NPTS)).reshape(NB, 3, NPTS)
    tp = jnp.matmul(R, pts_e) + t
    tp = jnp.transpose(tp, (0, 2, 1))
    x, y, z = tp[..., 0], tp[..., 1], tp[..., 2]
    normal_uv = jnp.stack([x / z, y / z, jnp.ones_like(x)], axis=-1)
    uv = jnp.matmul(normal_uv, jnp.transpose(K3, (0, 2, 1)))[:, :, :2]
    grid = (uv - 0.5).reshape(NB, NPTS, 1, 2)
    gx_ = grid[..., 0] / float(W - 1) * 2.0 - 1.0
    gy_ = grid[..., 1] / float(H - 1) * 2.0 - 1.0
    dx = 1.0 / float(W - 1) * 2.0
    dy = 1.0 / float(H - 1) * 2.0

    def to_ix(g):
        return (((g + 1.0) * W - 1.0) / 2.0)[:, :, 0]

    def to_iy(g):
        return (((g + 1.0) * H - 1.0) / 2.0)[:, :, 0]

    ixy = jnp.stack([to_ix(gx_), to_ix(gx_ - dx), to_ix(gx_ + dx),
                     to_iy(gy_), to_iy(gy_ - dy), to_iy(gy_ + dy)], axis=1)
    ixy = jnp.pad(ixy, ((0, 0), (0, 0), (0, NPAD - NPTS))).reshape(-1)

    pf8, gx8, gy8 = _sc_fetch(fm_pair, ixy)
    pf = pf8[:, :NPTS].reshape(B, V, C, NPTS)
    gx = gx8[:, :NPTS].reshape(B, V, C, NPTS)
    gy = gy8[:, :NPTS].reshape(B, V, C, NPTS)
    return pf, jnp.stack([gx, gy], axis=-1)


# speedup vs baseline: 6.3352x; 1.0364x over previous
import jax
import jax.numpy as jnp
from jax import lax
from jax.experimental import pallas as pl
from jax.experimental.pallas import tpu as pltpu
from jax.experimental.pallas import tpu_sc as plsc

B, V, C, H, W = 2, 4, 16, 256, 256
NB = B * V
NPTS = 50000
NPAD = 50176
QUART = NPAD // 4
CHUNK = 128
NCHUNK = QUART // CHUNK
NGROUP = CHUNK // 16

_SAMPLES = ((0, 0), (1, 0), (2, 0), (0, 1), (0, 2))


def _corner(f, sel_step):
    fc = jnp.minimum(jnp.maximum(f, -4.0), 300.0)
    ti = fc.astype(jnp.int32)
    tf = ti.astype(jnp.float32)
    neg = tf > fc
    f0 = jnp.where(neg, tf - 1.0, tf)
    i0 = jnp.where(neg, ti - 1, ti)
    w1 = fc - f0
    w0 = 1.0 - w1
    a0 = jnp.where((i0 >= 0) & (i0 <= W - 1), w0, 0.0)
    a1 = jnp.where((i0 >= -1) & (i0 <= W - 2), w1, 0.0)
    base = jnp.minimum(jnp.maximum(i0, 0), W - 2)
    match = i0 == base
    s0 = jnp.where(match, 0, sel_step)
    s1 = jnp.where(match, sel_step, 0)
    return base, (a0, a1), (s0, s1)


def _tec_body(fm_ref, ixy_ref, pf_ref, gx_ref, gy_ref,
              cxy_v, idx_v, w_v, cv_v, rows_v, pfb, gxb, gyb, sem0, sem1):
    cid = lax.axis_index("c")
    sid = lax.axis_index("s")
    wid = sid * 2 + cid
    vw = wid // 4
    qt = wid % 4
    vbase = vw * (H * W)
    sems = (sem0, sem1)

    def stage_fire(c, buf):
        base6 = (wid * NCHUNK + c) * (6 * CHUNK)
        ob6 = buf * 6 * CHUNK
        ob5 = buf * 5 * CHUNK
        ob20 = buf * 20 * CHUNK
        pltpu.sync_copy(ixy_ref.at[pl.ds(base6, 6 * CHUNK)],
                        cxy_v.at[pl.ds(ob6, 6 * CHUNK)])

        def p1(gi, c1):
            sl = gi * 16
            xs = [cxy_v[pl.ds(ob6 + r * CHUNK + sl, 16)] for r in (0, 1, 2)]
            ys = [cxy_v[pl.ds(ob6 + r * CHUNK + sl, 16)] for r in (3, 4, 5)]
            xd = [_corner(f, C) for f in xs]
            yd = [_corner(f, 2 * C) for f in ys]
            for s, (xv, yv) in enumerate(_SAMPLES):
                bx, (ax0, ax1), (sx0, sx1) = xd[xv]
                by, (ay0, ay1), (sy0, sy1) = yd[yv]
                idx_v[pl.ds(ob5 + s * CHUNK + sl, 16)] = by * W + bx + vbase
                for j, (ay, sy) in enumerate(((ay0, sy0), (ay1, sy1))):
                    for i, (ax, sx) in enumerate(((ax0, sx0), (ax1, sx1))):
                        kk = (s * 4 + j * 2 + i) * CHUNK
                        w_v[pl.ds(ob20 + kk + sl, 16)] = ax * ay
                        cv_v[pl.ds(ob20 + kk + sl, 16)] = sy + sx
            return c1

        lax.fori_loop(0, NGROUP, p1, 0)
        for s in range(5):
            pltpu.async_copy(
                fm_ref.at[idx_v.at[pl.ds(ob5 + s * CHUNK, CHUNK)]],
                rows_v.at[pl.ds(ob5 + s * CHUNK, CHUNK)], sems[buf])

    def drain_p2_out(c, buf):
        ob5 = buf * 5 * CHUNK
        ob20 = buf * 20 * CHUNK
        for s in range(5):
            pltpu.make_async_copy(
                fm_ref.at[idx_v.at[pl.ds(ob5 + s * CHUNK, CHUNK)]],
                rows_v.at[pl.ds(ob5 + s * CHUNK, CHUNK)], sems[buf]).wait()

        def p2(gi, c2):
            sl = gi * 16
            osl = pl.ds(sl, 16)
            ws = [w_v[pl.ds(ob20 + kk * CHUNK + sl, 16)] for kk in range(20)]
            cv = [cv_v[pl.ds(ob20 + kk * CHUNK + sl, 16)] for kk in range(20)]
            pvec = lax.iota(jnp.int32, 16) + sl
            rowv = [pvec + (ob5 + s * CHUNK) for s in range(5)]
            for ch in range(C):
                accs = []
                for s in range(5):
                    k0 = s * 4
                    a = ws[k0] * plsc.load_gather(
                        rows_v, [rowv[s], cv[k0] + ch])
                    for kk in (k0 + 1, k0 + 2, k0 + 3):
                        a = a + ws[kk] * plsc.load_gather(
                            rows_v, [rowv[s], cv[kk] + ch])
                    accs.append(a)
                pfb[ch, osl] = accs[0]
                gxb[ch, osl] = (accs[2] - accs[1]) * 0.5
                gyb[ch, osl] = (accs[4] - accs[3]) * 0.5
            return c2

        lax.fori_loop(0, NGROUP, p2, 0)
        pbase = qt * QUART + c * CHUNK
        rsl = (pl.ds(vw * C, C), pl.ds(pbase, CHUNK))
        pltpu.sync_copy(pfb, pf_ref.at[rsl])
        pltpu.sync_copy(gxb, gx_ref.at[rsl])
        pltpu.sync_copy(gyb, gy_ref.at[rsl])

    stage_fire(0, 0)

    def pair_body(i, carry):
        a = 2 * i
        stage_fire(a + 1, 1)
        drain_p2_out(a, 0)
        pl.when(i < NCHUNK // 2 - 1)(lambda: stage_fire(a + 2, 0))
        drain_p2_out(a + 1, 1)
        return carry

    lax.fori_loop(0, NCHUNK // 2, pair_body, 0)


_out3 = [jax.ShapeDtypeStruct((NB * C, NPAD), jnp.float32)] * 3

_sc_fetch = pl.kernel(
    _tec_body,
    _out3,
    mesh=plsc.VectorSubcoreMesh(core_axis_name="c", subcore_axis_name="s",
                                num_cores=2, num_subcores=16),
    scratch_types=[
        pltpu.VMEM((2 * 6 * CHUNK,), jnp.float32),
        pltpu.VMEM((2 * 5 * CHUNK,), jnp.int32),
        pltpu.VMEM((2 * 20 * CHUNK,), jnp.float32),
        pltpu.VMEM((2 * 20 * CHUNK,), jnp.int32),
        pltpu.VMEM((2 * 5 * CHUNK, 4 * C), jnp.float32),
        pltpu.VMEM((C, CHUNK), jnp.float32),
        pltpu.VMEM((C, CHUNK), jnp.float32),
        pltpu.VMEM((C, CHUNK), jnp.float32),
        pltpu.SemaphoreType.DMA,
        pltpu.SemaphoreType.DMA,
    ],
    compiler_params=pltpu.CompilerParams(needs_layout_passes=False,
                                         use_tc_tiling_on_sc=False),
)


def kernel(feature_maps, pts, cam_intrinsics, cam_extrinsics):
    a = jnp.transpose(feature_maps.reshape(NB, C, H * W), (0, 2, 1))

    def shifted(k):
        return jnp.concatenate([a[:, k:, :], a[:, :k, :]], axis=1)

    quad = jnp.concatenate([a, shifted(1), shifted(W), shifted(W + 1)],
                           axis=2).reshape(NB * H * W, 4 * C)

    K3 = cam_intrinsics.reshape(NB, 3, 3)
    E3 = cam_extrinsics.reshape(NB, 3, 4)
    R = E3[:, :, 0:3]
    t = E3[:, :, 3:4]
    pts_e = jnp.broadcast_to(pts[:, None, :, :],
                             (B, V, 3, NPTS)).reshape(NB, 3, NPTS)
    tp = jnp.matmul(R, pts_e) + t
    tp = jnp.transpose(tp, (0, 2, 1))
    x, y, z = tp[..., 0], tp[..., 1], tp[..., 2]
    normal_uv = jnp.stack([x / z, y / z, jnp.ones_like(x)], axis=-1)
    uv = jnp.matmul(normal_uv, jnp.transpose(K3, (0, 2, 1)))[:, :, :2]
    grid = (uv - 0.5).reshape(NB, NPTS, 1, 2)
    gx_ = grid[..., 0] / float(W - 1) * 2.0 - 1.0
    gy_ = grid[..., 1] / float(H - 1) * 2.0 - 1.0
    dx = 1.0 / float(W - 1) * 2.0
    dy = 1.0 / float(H - 1) * 2.0

    def to_ix(g):
        return (((g + 1.0) * W - 1.0) / 2.0)[:, :, 0]

    def to_iy(g):
        return (((g + 1.0) * H - 1.0) / 2.0)[:, :, 0]

    ixy = jnp.stack([to_ix(gx_), to_ix(gx_ - dx), to_ix(gx_ + dx),
                     to_iy(gy_), to_iy(gy_ - dy), to_iy(gy_ + dy)], axis=1)
    ixy = jnp.pad(ixy, ((0, 0), (0, 0), (0, NPAD - NPTS)))
    ixy = ixy.reshape(NB, 6, 4, NCHUNK, CHUNK).transpose(0, 2, 3, 1, 4)
    ixy = ixy.reshape(-1)

    pf8, gx8, gy8 = _sc_fetch(quad, ixy)
    pf = pf8[:, :NPTS].reshape(B, V, C, NPTS)
    gx = gx8[:, :NPTS].reshape(B, V, C, NPTS)
    gy = gy8[:, :NPTS].reshape(B, V, C, NPTS)
    return pf, jnp.stack([gx, gy], axis=-1)
